# baseline (speedup 1.0000x reference)
import functools

import jax
import jax.numpy as jnp
from jax import lax
from jax.experimental import pallas as pl
from jax.experimental.pallas import tpu as pltpu
from jax.experimental.pallas import tpu_sc as plsc

B = 4096
L = 200
D = 64
V = 1000
NC = 2
NS = 16
NW = NC * NS
LT = L // 8
BT = B // 128
PH = 2
DH = D // PH


def _body(ridx4, cidx4, rowt_hbm, colt_hbm, out5,
          tstage, rowT_v, colT_v, rtile, ctile, obuf, semout):
    bt = lax.axis_index("s") * NC + lax.axis_index("c")
    lanes = lax.iota(jnp.int32, 16)

    def load_table_half(tbl_hbm, dstT, ph):
        pltpu.sync_copy(tbl_hbm.at[pl.ds(0, V), pl.ds(ph * DH, DH)], tstage)

        @plsc.parallel_loop(0, V)
        def trow(e):
            ev = jnp.full((16,), e, jnp.int32)
            for g in range(DH // 16):
                v = tstage[e, pl.ds(g * 16, 16)]
                plsc.store_scatter(dstT, [lanes + (g * 16), ev], v)

    def out_cp(l, ph, p):
        return pltpu.make_async_copy(
            obuf.at[p],
            out5.at[l, pl.ds(ph * (DH // 8), DH // 8), bt], semout)

    for ph in range(PH):
        load_table_half(rowt_hbm, rowT_v, ph)
        load_table_half(colt_hbm, colT_v, ph)

        def ltile(tr, carry):
            pltpu.sync_copy(ridx4.at[tr, bt], rtile)
            pltpu.sync_copy(cidx4.at[tr, bt], ctile)

            def lrow(r, c2):
                l = tr * 8 + r
                p = r & 1

                @pl.when(l >= 2)
                def _drain_prev():
                    out_cp(0, 0, 0).wait()

                @plsc.parallel_loop(0, 8, unroll=2)
                def lane_group(g):
                    i16r = rtile[r, pl.ds(g * 16, 16)]
                    i16c = ctile[r, pl.ds(g * 16, 16)]
                    for dl in range(DH):
                        dv = jnp.full((16,), dl, jnp.int32)
                        v = (plsc.load_gather(rowT_v, [dv, i16r]) +
                             plsc.load_gather(colT_v, [dv, i16c]))
                        obuf[p, dl // 8, dl % 8, pl.ds(g * 16, 16)] = v

                out_cp(l, ph, p).start()
                return c2

            lax.fori_loop(0, 8, lrow, 0)
            return carry

        lax.fori_loop(0, LT, ltile, 0)
        out_cp(0, 0, 0).wait()
        out_cp(0, 0, 0).wait()


@jax.jit
def kernel(row_indices, col_indices, row_table, col_table):
    ridx4 = jnp.transpose(
        row_indices.astype(jnp.int32).T.reshape(LT, 8, BT, 128), (0, 2, 1, 3))
    cidx4 = jnp.transpose(
        col_indices.astype(jnp.int32).T.reshape(LT, 8, BT, 128), (0, 2, 1, 3))
    k = pl.kernel(
        _body,
        mesh=plsc.VectorSubcoreMesh(core_axis_name="c", subcore_axis_name="s"),
        compiler_params=pltpu.CompilerParams(
            use_tc_tiling_on_sc=False, needs_layout_passes=False),
        out_type=jax.ShapeDtypeStruct((L, D // 8, BT, 8, 128), jnp.float32),
        scratch_types=[
            pltpu.VMEM((V, DH), jnp.float32),
            pltpu.VMEM((DH, V), jnp.float32),
            pltpu.VMEM((DH, V), jnp.float32),
            pltpu.VMEM((8, 128), jnp.int32),
            pltpu.VMEM((8, 128), jnp.int32),
            pltpu.VMEM((2, DH // 8, 8, 128), jnp.float32),
            pltpu.SemaphoreType.DMA,
        ],
    )
    out5 = k(ridx4, cidx4, row_table, col_table)
    return out5.transpose(2, 4, 0, 1, 3).reshape(B, L, D)


# speedup vs baseline: 1.1282x; 1.1282x over previous
import functools

import jax
import jax.numpy as jnp
from jax import lax
from jax.experimental import pallas as pl
from jax.experimental.pallas import tpu as pltpu
from jax.experimental.pallas import tpu_sc as plsc

B = 4096
L = 200
D = 64
V = 1000
NC = 2
NS = 16
NW = NC * NS
LT = L // 8
BT = B // 128
PH = 2
DH = D // PH


def _body(ridx4, cidx4, rowt_hbm, colt_hbm, out5,
          tstage, rowT_v, colT_v, rtile, ctile, obuf, semout):
    bt = lax.axis_index("s") * NC + lax.axis_index("c")
    lanes = lax.iota(jnp.int32, 16)

    def load_table_half(tbl_hbm, dstT, ph):
        pltpu.sync_copy(tbl_hbm.at[pl.ds(0, V), pl.ds(ph * DH, DH)], tstage)

        @plsc.parallel_loop(0, V)
        def trow(e):
            ev = jnp.full((16,), e, jnp.int32)
            for g in range(DH // 16):
                v = tstage[e, pl.ds(g * 16, 16)]
                plsc.store_scatter(dstT, [lanes + (g * 16), ev], v)

    def out_cp(l, ph, p):
        return pltpu.make_async_copy(
            obuf.at[p],
            out5.at[l, pl.ds(ph * (DH // 8), DH // 8), bt], semout)

    for ph in range(PH):
        load_table_half(rowt_hbm, rowT_v, ph)
        load_table_half(colt_hbm, colT_v, ph)

        def ltile(tr, carry):
            pltpu.sync_copy(ridx4.at[tr, bt], rtile)
            pltpu.sync_copy(cidx4.at[tr, bt], ctile)

            def lrow(r, c2):
                l = tr * 8 + r
                p = r & 1

                @pl.when(l >= 2)
                def _drain_prev():
                    out_cp(0, 0, 0).wait()

                @plsc.parallel_loop(0, 8)
                def lane_group(g):
                    i16r = rtile[r, pl.ds(g * 16, 16)]
                    i16c = ctile[r, pl.ds(g * 16, 16)]
                    for dl in range(DH):
                        dv = jnp.full((16,), dl, jnp.int32)
                        v = (plsc.load_gather(rowT_v, [dv, i16r]) +
                             plsc.load_gather(colT_v, [dv, i16c]))
                        obuf[p, dl // 8, dl % 8, pl.ds(g * 16, 16)] = v

                out_cp(l, ph, p).start()
                return c2

            lax.fori_loop(0, 8, lrow, 0)
            return carry

        lax.fori_loop(0, LT, ltile, 0)
        out_cp(0, 0, 0).wait()
        out_cp(0, 0, 0).wait()


@jax.jit
def kernel(row_indices, col_indices, row_table, col_table):
    ridx4 = jnp.transpose(
        row_indices.astype(jnp.int32).T.reshape(LT, 8, BT, 128), (0, 2, 1, 3))
    cidx4 = jnp.transpose(
        col_indices.astype(jnp.int32).T.reshape(LT, 8, BT, 128), (0, 2, 1, 3))
    k = pl.kernel(
        _body,
        mesh=plsc.VectorSubcoreMesh(core_axis_name="c", subcore_axis_name="s"),
        compiler_params=pltpu.CompilerParams(
            use_tc_tiling_on_sc=False, needs_layout_passes=False),
        out_type=jax.ShapeDtypeStruct((L, D // 8, BT, 8, 128), jnp.float32),
        scratch_types=[
            pltpu.VMEM((V, DH), jnp.float32),
            pltpu.VMEM((DH, V), jnp.float32),
            pltpu.VMEM((DH, V), jnp.float32),
            pltpu.VMEM((8, 128), jnp.int32),
            pltpu.VMEM((8, 128), jnp.int32),
            pltpu.VMEM((2, DH // 8, 8, 128), jnp.float32),
            pltpu.SemaphoreType.DMA,
        ],
    )
    out5 = k(ridx4, cidx4, row_table, col_table)
    return out5.transpose(2, 4, 0, 1, 3).reshape(B, L, D)
